# baseline (speedup 1.0000x reference)
import functools

import jax
import jax.numpy as jnp
from jax.experimental import pallas as pl
from jax.experimental.pallas import tpu as pltpu

TEMP = 0.1
EPS = 1e-8
LOG2E = 1.4426950408889634
BM = 256
CHUNK = 256
TCH = 640
NEG_BIG = -30000.0


def _loss_block_kernel(pe_full_ref, at_ref, et_ref, out_ref, e_scr):
    bsz, c, hw = pe_full_ref.shape
    n = bsz * hw

    @pl.when(pl.program_id(0) == 0)
    def _build_e():
        for bi in range(bsz):
            for off in range(0, hw, TCH):
                blk = pe_full_ref[bi, :, off:off + TCH]
                e_scr[bi * hw + off:bi * hw + off + TCH, :] = (
                    jnp.transpose(blk, (1, 0)).astype(jnp.bfloat16))

    scale = jnp.float32(LOG2E / TEMP)
    a = (at_ref[0] * scale).astype(jnp.bfloat16)
    af = a.astype(jnp.float32)
    pos = jnp.sum(af * et_ref[0], axis=0, keepdims=True)

    m16 = jnp.full((16, BM), NEG_BIG, jnp.bfloat16)
    d16 = jnp.zeros((16, BM), jnp.float32)
    for k in range(0, n, CHUNK):
        s_k = jnp.dot(e_scr[k:k + CHUNK, :], a,
                      preferred_element_type=jnp.float32)
        sb3 = s_k.astype(jnp.bfloat16).reshape(CHUNK // 16, 16, BM)
        m16n = jnp.maximum(m16, jnp.max(sb3, axis=0))
        t = jnp.exp2(sb3 - m16n[None, :, :])
        while t.shape[0] > 1:
            half = t.shape[0] // 2
            t = t[:half] + t[half:]
        d16 = (d16 * jnp.exp2(m16.astype(jnp.float32)
                              - m16n.astype(jnp.float32))
               + t[0].astype(jnp.float32))
        m16 = m16n
    m16f = m16.astype(jnp.float32)
    m_rel = jnp.max(m16f, axis=0, keepdims=True)
    d = jnp.sum(d16 * jnp.exp2(m16f - m_rel), axis=0, keepdims=True)
    p = jnp.exp2(pos - m_rel)
    ratio = p / (d + p + jnp.float32(EPS))
    out_ref[0] = -jnp.log(ratio + jnp.float32(EPS))


@jax.jit
def _contrastive_loss(proj_main, proj_ema):
    b, c, H, W = proj_main.shape
    N = b * H * W
    hw = H * W
    pb = hw // BM
    pm3 = proj_main.reshape(b, c, hw)
    pe3 = proj_ema.reshape(b, c, hw)
    grid = (N // BM,)
    losses = pl.pallas_call(
        _loss_block_kernel,
        grid=grid,
        in_specs=[
            pl.BlockSpec((b, c, hw), lambda i: (0, 0, 0)),
            pl.BlockSpec((1, c, BM), lambda i: (i // pb, 0, i % pb)),
            pl.BlockSpec((1, c, BM), lambda i: (i // pb, 0, i % pb)),
        ],
        out_specs=pl.BlockSpec((1, 1, BM), lambda i: (i, 0, 0)),
        out_shape=jax.ShapeDtypeStruct((N // BM, 1, BM), jnp.float32),
        scratch_shapes=[pltpu.VMEM((N, c), jnp.bfloat16)],
        compiler_params=pltpu.CompilerParams(
            dimension_semantics=("arbitrary",),
            vmem_limit_bytes=100 * 1024 * 1024,
        ),
    )(pe3, pm3, pe3)
    return jnp.mean(losses)


def kernel(proj_main, proj_ema, label_main, label_ema, patch_num):
    return _contrastive_loss(proj_main, proj_ema)


# speedup vs baseline: 1.1186x; 1.1186x over previous
import functools

import jax
import jax.numpy as jnp
from jax.experimental import pallas as pl
from jax.experimental.pallas import tpu as pltpu

TEMP = 0.1
EPS = 1e-8
LOG2E = 1.4426950408889634
BM = 640
CHUNK = 256
TCH = 640
NEG_BIG = -30000.0


def _loss_block_kernel(pe_full_ref, at_ref, et_ref, out_ref, e_scr):
    bsz, c, hw = pe_full_ref.shape
    n = bsz * hw

    @pl.when(pl.program_id(0) == 0)
    def _build_e():
        for bi in range(bsz):
            for off in range(0, hw, TCH):
                blk = pe_full_ref[bi, :, off:off + TCH]
                e_scr[bi * hw + off:bi * hw + off + TCH, :] = (
                    jnp.transpose(blk, (1, 0)).astype(jnp.bfloat16))

    scale = jnp.float32(LOG2E / TEMP)
    a = (at_ref[0] * scale).astype(jnp.bfloat16)
    af = a.astype(jnp.float32)
    pos = jnp.sum(af * et_ref[0], axis=0, keepdims=True)

    m8 = jnp.full((8, BM), NEG_BIG, jnp.float32)
    d8 = jnp.zeros((8, BM), jnp.float32)
    for k in range(0, n, CHUNK):
        s_k = jnp.dot(e_scr[k:k + CHUNK, :], a,
                      preferred_element_type=jnp.float32)
        s3 = s_k.reshape(CHUNK // 8, 8, BM)
        m8n = jnp.maximum(m8, jnp.max(s3, axis=0))
        p3 = jnp.exp2(s3 - m8n[None, :, :])
        d8 = d8 * jnp.exp2(m8 - m8n) + jnp.sum(p3, axis=0)
        m8 = m8n
    m_rel = jnp.max(m8, axis=0, keepdims=True)
    d = jnp.sum(d8 * jnp.exp2(m8 - m_rel), axis=0, keepdims=True)
    p = jnp.exp2(pos - m_rel)
    ratio = p / (d + p + jnp.float32(EPS))
    out_ref[0] = -jnp.log(ratio + jnp.float32(EPS))


@jax.jit
def _contrastive_loss(proj_main, proj_ema):
    b, c, H, W = proj_main.shape
    N = b * H * W
    hw = H * W
    pb = hw // BM
    pm3 = proj_main.reshape(b, c, hw)
    pe3 = proj_ema.reshape(b, c, hw)
    grid = (N // BM,)
    losses = pl.pallas_call(
        _loss_block_kernel,
        grid=grid,
        in_specs=[
            pl.BlockSpec((b, c, hw), lambda i: (0, 0, 0)),
            pl.BlockSpec((1, c, BM), lambda i: (i // pb, 0, i % pb)),
            pl.BlockSpec((1, c, BM), lambda i: (i // pb, 0, i % pb)),
        ],
        out_specs=pl.BlockSpec((1, 1, BM), lambda i: (i, 0, 0)),
        out_shape=jax.ShapeDtypeStruct((N // BM, 1, BM), jnp.float32),
        scratch_shapes=[pltpu.VMEM((N, c), jnp.bfloat16)],
        compiler_params=pltpu.CompilerParams(
            dimension_semantics=("arbitrary",),
            vmem_limit_bytes=100 * 1024 * 1024,
        ),
    )(pe3, pm3, pe3)
    return jnp.mean(losses)


def kernel(proj_main, proj_ema, label_main, label_ema, patch_num):
    return _contrastive_loss(proj_main, proj_ema)


# speedup vs baseline: 1.1264x; 1.0070x over previous
import functools

import jax
import jax.numpy as jnp
from jax.experimental import pallas as pl
from jax.experimental.pallas import tpu as pltpu

TEMP = 0.1
EPS = 1e-8
LOG2E = 1.4426950408889634
BM = 1280
CHUNK = 256
TCH = 640
NEG_BIG = -30000.0


def _loss_block_kernel(pe_full_ref, at_ref, et_ref, out_ref, e_scr):
    bsz, c, hw = pe_full_ref.shape
    n = bsz * hw

    @pl.when(pl.program_id(0) == 0)
    def _build_e():
        for bi in range(bsz):
            for off in range(0, hw, TCH):
                blk = pe_full_ref[bi, :, off:off + TCH]
                e_scr[bi * hw + off:bi * hw + off + TCH, :] = (
                    jnp.transpose(blk, (1, 0)).astype(jnp.bfloat16))

    scale = jnp.float32(LOG2E / TEMP)
    a = (at_ref[0] * scale).astype(jnp.bfloat16)
    af = a.astype(jnp.float32)
    pos = jnp.sum(af * et_ref[0], axis=0, keepdims=True)

    m8 = jnp.full((8, BM), NEG_BIG, jnp.float32)
    d8 = jnp.zeros((8, BM), jnp.float32)
    for k in range(0, n, CHUNK):
        s_k = jnp.dot(e_scr[k:k + CHUNK, :], a,
                      preferred_element_type=jnp.float32)
        s3 = s_k.reshape(CHUNK // 8, 8, BM)
        m8n = jnp.maximum(m8, jnp.max(s3, axis=0))
        p3 = jnp.exp2(s3 - m8n[None, :, :])
        d8 = d8 * jnp.exp2(m8 - m8n) + jnp.sum(p3, axis=0)
        m8 = m8n
    m_rel = jnp.max(m8, axis=0, keepdims=True)
    d = jnp.sum(d8 * jnp.exp2(m8 - m_rel), axis=0, keepdims=True)
    p = jnp.exp2(pos - m_rel)
    ratio = p / (d + p + jnp.float32(EPS))
    out_ref[0] = -jnp.log(ratio + jnp.float32(EPS))


@jax.jit
def _contrastive_loss(proj_main, proj_ema):
    b, c, H, W = proj_main.shape
    N = b * H * W
    hw = H * W
    pb = hw // BM
    pm3 = proj_main.reshape(b, c, hw)
    pe3 = proj_ema.reshape(b, c, hw)
    grid = (N // BM,)
    losses = pl.pallas_call(
        _loss_block_kernel,
        grid=grid,
        in_specs=[
            pl.BlockSpec((b, c, hw), lambda i: (0, 0, 0)),
            pl.BlockSpec((1, c, BM), lambda i: (i // pb, 0, i % pb)),
            pl.BlockSpec((1, c, BM), lambda i: (i // pb, 0, i % pb)),
        ],
        out_specs=pl.BlockSpec((1, 1, BM), lambda i: (i, 0, 0)),
        out_shape=jax.ShapeDtypeStruct((N // BM, 1, BM), jnp.float32),
        scratch_shapes=[pltpu.VMEM((N, c), jnp.bfloat16)],
        compiler_params=pltpu.CompilerParams(
            dimension_semantics=("arbitrary",),
            vmem_limit_bytes=100 * 1024 * 1024,
        ),
    )(pe3, pm3, pe3)
    return jnp.mean(losses)


def kernel(proj_main, proj_ema, label_main, label_ema, patch_num):
    return _contrastive_loss(proj_main, proj_ema)
